# baseline (speedup 1.0000x reference)
import functools

import jax
import jax.numpy as jnp
from jax import lax
from jax.experimental import pallas as pl
from jax.experimental.pallas import tpu as pltpu
from jax.experimental.pallas import tpu_sc as plsc

_B, _S, _D = 4, 4096, 768
_NC, _NS = 2, 16
_NW = _NC * _NS
_SPW = _S // _NW
_CH = 32
_NSUB = _SPW // _CH
_BP = 2
_NPAIR = _B // _BP
_NCHUNK = _NSUB * _NPAIR
_ROWS = _BP * _CH
_LANES = 16
_JV = _D // _LANES


def _make_emb_kernel():
    mesh = plsc.VectorSubcoreMesh(core_axis_name="c", subcore_axis_name="s")

    @functools.partial(
        pl.kernel,
        mesh=mesh,
        out_type=jax.ShapeDtypeStruct((_B, _S, _D), jnp.float32),
        scratch_types=[
            pltpu.VMEM((_NCHUNK, _ROWS), jnp.int32),
            pltpu.VMEM((2, _ROWS, _D), jnp.float32),
            pltpu.VMEM((_CH, _D), jnp.float32),
            pltpu.SemaphoreType.DMA,
            pltpu.SemaphoreType.DMA,
            pltpu.SemaphoreType.DMA,
            pltpu.SemaphoreType.DMA,
            pltpu.SemaphoreType.DMA,
            pltpu.SemaphoreType.DMA,
        ],
    )
    def emb(x_hbm, table_hbm, pe_hbm, out_hbm,
            idx_v, rows_v, pe_v, sem_i, sem_pe,
            sem_g0, sem_g1, sem_o0, sem_o1):
        wid = lax.axis_index("s") * _NC + lax.axis_index("c")
        s_base = wid * _SPW
        sems_g = (sem_g0, sem_g1)
        sems_o = (sem_o0, sem_o1)

        def coords(t):
            sub = t // _NPAIR
            b0 = (t % _NPAIR) * _BP
            s0 = s_base + sub * _CH
            return sub, b0, s0

        def pe_desc(sub):
            return pltpu.make_async_copy(
                pe_hbm.at[pl.ds(s_base + sub * _CH, _CH)], pe_v, sem_pe)

        def idx_desc(t, i):
            _, b0, s0 = coords(t)
            return pltpu.make_async_copy(
                x_hbm.at[b0 + i, pl.ds(s0, _CH)],
                idx_v.at[t, pl.ds(i * _CH, _CH)], sem_i)

        def gather_desc(t, k):
            return pltpu.make_async_copy(table_hbm.at[idx_v.at[t]],
                                         rows_v.at[k], sems_g[k])

        def out_desc(t, k, i):
            _, b0, s0 = coords(t)
            return pltpu.make_async_copy(
                rows_v.at[k, pl.ds(i * _CH, _CH), :],
                out_hbm.at[b0 + i, pl.ds(s0, _CH), :], sems_o[k])

        for t in range(_NCHUNK):
            for i in range(_BP):
                idx_desc(t, i).start()
        pe_desc(0).start()
        for t in range(_NCHUNK):
            for i in range(_BP):
                idx_desc(t, i).wait()

        @pl.loop(0, _NCHUNK, step=2)
        def _chunks(c):
            for k in range(2):
                t = c + k
                if k == 0:
                    pe_desc(0).wait()

                @pl.when(c > 0)
                def _():
                    tp = lax.max(t - 2, 0)
                    for i in range(_BP):
                        out_desc(tp, k, i).wait()

                gather_desc(t, k).start()

            for k in range(2):
                t = c + k
                sub, b0, s0 = coords(t)
                gather_desc(t, k).wait()

                def row_body(r, carry):
                    for j in range(_JV):
                        sl = pl.ds(j * _LANES, _LANES)
                        v = pe_v[r, sl]
                        plsc.addupdate(rows_v.at[k, r, sl], v)
                        plsc.addupdate(rows_v.at[k, _CH + r, sl], v)
                    return carry

                lax.fori_loop(0, _CH, row_body, 0)
                for i in range(_BP):
                    out_desc(t, k, i).start()
                if k == 1:

                    @pl.when(c < _NCHUNK - 2)
                    def _():
                        pe_desc(lax.min(sub + 1, _NSUB - 1)).start()

        for k in range(2):
            for i in range(_BP):
                out_desc(_NCHUNK - 2 + k, k, i).wait()

    return emb


_emb = _make_emb_kernel()


def kernel(x, table, pe):
    return _emb(x.astype(jnp.int32), table, pe)
